# baseline (speedup 1.0000x reference)
import jax
import jax.numpy as jnp
from jax import lax
from jax.experimental import pallas as pl
from jax.experimental.pallas import tpu as pltpu
from jax.experimental.pallas import tpu_sc as plsc

N = 10000
E = 640000
NPAD = 10240
NS = 16
NC = 2
NSL = NPAD // NS
W = 125
ROWS = E // W
G = ROWS // (NS * NC)
EH = E // (NS * NC)
WH = 2000


def _body(er, rowf, colf, xf, prm, out, dex, aex1, aex2,
          deg_s, g_s, agg_s, hf_s,
          rb, cb, pay0, pay1, ones, xb, pb, degb, degc, dvb, hb, gb, ab, a2,
          hfb, hfl, rbh, cbh, ob, sem0, sem1, sem2, sem3, semc):
    sid = lax.axis_index("s")
    cid = lax.axis_index("c")
    oid = 1 - cid
    wid = sid * NC + cid
    ns0 = sid * NSL
    f32 = jnp.float32

    def bf16r(v):
        u = lax.bitcast_convert_type(v, jnp.uint32)
        r = u + jnp.uint32(0x7FFF) + ((u >> jnp.uint32(16)) & jnp.uint32(1))
        return lax.bitcast_convert_type(r & jnp.uint32(0xFFFF0000), f32)

    z16 = jnp.zeros((16,), f32)

    def _za(i, _):
        ab[i, :] = z16
        return 0
    lax.fori_loop(0, NSL, _za, 0)

    def _zb(i, _):
        hfb[pl.ds(i * 16, 16)] = z16
        return 0
    lax.fori_loop(0, NSL // 16, _zb, 0)

    one16 = jnp.ones((16,), f32)
    for j in range(128 // 16):
        ones[pl.ds(j * 16, 16)] = one16

    pltpu.sync_copy(prm, pb)

    @pl.when(sid < NS - 1)
    def _():
        pltpu.sync_copy(xf.at[pl.ds(sid * (2 * NSL), 2 * NSL)], xb)

    @pl.when(sid == NS - 1)
    def _():
        rem = 2 * N - (NS - 1) * (2 * NSL)
        pltpu.sync_copy(xf.at[pl.ds((NS - 1) * (2 * NSL), rem)],
                        xb.at[pl.ds(0, rem)])

    pltpu.sync_copy(ab, agg_s.at[pl.ds(ns0, NSL)])
    pltpu.sync_copy(hfb, deg_s.at[pl.ds(ns0, NSL)])
    plsc.subcore_barrier()

    sems = (sem0, sem1)
    with jax.named_scope("ph_hist"):
        r0 = wid * G
        pltpu.sync_copy(er.at[1, pl.ds(r0, G), :], cb)

        def _bpair(k, _):
            for b in range(2):
                i = 2 * k + b

                @pl.when(k > 0)
                def _():
                    pltpu.make_async_copy(
                        ones.at[pl.ds(0, W)], deg_s.at[cb.at[i]],
                        sems[b]).wait()
                pltpu.async_copy(ones.at[pl.ds(0, W)],
                                 deg_s.at[cb.at[i]], sems[b], add=True)
            return 0
        lax.fori_loop(0, G // 2, _bpair, 0)
        for b in range(2):
            pltpu.make_async_copy(ones.at[pl.ds(0, W)],
                                  deg_s.at[cb.at[b]], sems[b]).wait()
    plsc.subcore_barrier()
    pltpu.sync_copy(deg_s.at[pl.ds(ns0, NSL)], dex.at[cid, pl.ds(ns0, NSL)])
    pltpu.core_barrier(semc, core_axis_name="c")
    pltpu.sync_copy(dex.at[oid, pl.ds(ns0, NSL)], degc)

    pltpu.sync_copy(deg_s.at[pl.ds(ns0, NSL)], degb)
    w1r0 = bf16r(pb[pl.ds(0, 16)])
    w1r1 = bf16r(pb[pl.ds(16, 16)])

    def _rsq(i, _):
        d = degb[pl.ds(i * 16, 16)] + degc[pl.ds(i * 16, 16)] + 2.0
        ii = lax.bitcast_convert_type(d, jnp.int32)
        ii = jnp.int32(0x5F3759DF) - (ii >> 1)
        y = lax.bitcast_convert_type(ii, f32)
        y = y * (1.5 - 0.5 * d * y * y)
        y = y * (1.5 - 0.5 * d * y * y)
        y = y * (1.5 - 0.5 * d * y * y)
        y = y * (1.5 - 0.5 * d * y * y)
        dvb[pl.ds(i * 16, 16)] = y
        return 0
    lax.fori_loop(0, NSL // 16, _rsq, 0)

    def _node1(i, _):
        dv16 = dvb[pl.ds(i * 16, 16)]
        xv0 = bf16r(xb[pl.ds(i * 32, 16)])
        xv1 = bf16r(xb[pl.ds(i * 32 + 16, 16)])
        for j in range(16):
            xv = xv0 if j < 8 else xv1
            xa = xv[(2 * j) % 16]
            xc = xv[(2 * j + 1) % 16]
            h1 = xa * w1r0 + xc * w1r1
            n = i * 16 + j
            hb[n, :] = h1
            gb[n, :] = dv16[j] * h1
        return 0
    lax.fori_loop(0, NSL // 16, _node1, 0)
    pltpu.sync_copy(gb, g_s.at[pl.ds(ns0, NSL)])
    plsc.subcore_barrier()

    pays = (pay0, pay1)

    def _agg_pass():
        r0 = wid * G
        pltpu.sync_copy(er.at[0, pl.ds(r0, G), :], rb)
        pltpu.sync_copy(er.at[1, pl.ds(r0, G), :], cb)

        semg = (sem2, sem3)
        pltpu.async_copy(g_s.at[rb.at[0]], pay0, semg[0])

        def _dpair(k, _):
            for b in range(2):
                i = 2 * k + b
                pltpu.make_async_copy(g_s.at[rb.at[i]], pays[b],
                                      semg[b]).wait()
                nb = 1 - b

                @pl.when(i > 0)
                def _():
                    pltpu.make_async_copy(
                        pays[nb], agg_s.at[cb.at[i]], sems[nb]).wait()

                @pl.when(i < G - 1)
                def _():
                    pltpu.async_copy(g_s.at[rb.at[i + 1]], pays[nb],
                                     semg[nb])
                pltpu.async_copy(pays[b], agg_s.at[cb.at[i]], sems[b],
                                 add=True)
            return 0
        lax.fori_loop(0, G // 2, _dpair, 0)
        lb = (G - 1) % 2
        pltpu.make_async_copy(pays[lb], agg_s.at[cb.at[0]],
                              sems[lb]).wait()

    with jax.named_scope("ph_agg1"):
        _agg_pass()
    plsc.subcore_barrier()
    pltpu.sync_copy(agg_s.at[pl.ds(ns0, NSL)], aex1.at[cid, pl.ds(ns0, NSL)])
    pltpu.core_barrier(semc, core_axis_name="c")
    pltpu.sync_copy(aex1.at[oid, pl.ds(ns0, NSL)], a2)

    pltpu.sync_copy(agg_s.at[pl.ds(ns0, NSL)], ab)
    b1v = pb[pl.ds(32, 16)]
    w2r = [bf16r(pb[pl.ds(48 + 16 * k, 16)]) for k in range(16)]

    def _node2(i, _):
        dv16 = dvb[pl.ds(i * 16, 16)]
        for j in range(16):
            n = i * 16 + j
            dv = dv16[j]
            z1 = jnp.maximum(
                dv * (ab[n, :] + a2[n, :]) + (2.0 * dv * dv) * hb[n, :] + b1v,
                0.0)
            ab[n, :] = z16
            z1 = bf16r(z1)
            acc = z1[0] * w2r[0]
            for k in range(1, 16):
                acc = acc + z1[k] * w2r[k]
            hb[n, :] = acc
            gb[n, :] = dv * acc
        return 0
    lax.fori_loop(0, NSL // 16, _node2, 0)
    pltpu.sync_copy(gb, g_s.at[pl.ds(ns0, NSL)])
    pltpu.sync_copy(ab, agg_s.at[pl.ds(ns0, NSL)])
    plsc.subcore_barrier()

    with jax.named_scope("ph_agg2"):
        _agg_pass()
    plsc.subcore_barrier()
    pltpu.sync_copy(agg_s.at[pl.ds(ns0, NSL)], aex2.at[cid, pl.ds(ns0, NSL)])
    pltpu.core_barrier(semc, core_axis_name="c")
    pltpu.sync_copy(aex2.at[oid, pl.ds(ns0, NSL)], a2)

    pltpu.sync_copy(agg_s.at[pl.ds(ns0, NSL)], ab)
    b2v = pb[pl.ds(304, 16)]
    wfv = bf16r(pb[pl.ds(320, 16)])
    wfs = [wfv[k] for k in range(16)]
    bf0 = pb[pl.ds(336, 16)][0]
    lane = lax.iota(jnp.int32, 16)

    def _node3(i, _):
        dv16 = dvb[pl.ds(i * 16, 16)]
        hfv = jnp.zeros((16,), f32)
        for j in range(16):
            n = i * 16 + j
            dv = dv16[j]
            z2 = jnp.maximum(
                dv * (ab[n, :] + a2[n, :]) + (2.0 * dv * dv) * hb[n, :] + b2v,
                0.0)
            z2 = bf16r(z2)
            dot = bf0
            for k in range(16):
                dot = dot + wfs[k] * z2[k]
            hfv = jnp.where(lane == j, dot, hfv)
        hfb[pl.ds(i * 16, 16)] = hfv
        return 0
    lax.fori_loop(0, NSL // 16, _node3, 0)
    pltpu.sync_copy(hfb, hf_s.at[pl.ds(ns0, NSL)])
    plsc.subcore_barrier()

    with jax.named_scope("ph_edge_out"):
        pltpu.sync_copy(hf_s, hfl)
        for gi in range(EH // WH):
            e0 = wid * EH + gi * WH
            pltpu.sync_copy(rowf.at[pl.ds(e0, WH)], rbh)
            pltpu.sync_copy(colf.at[pl.ds(e0, WH)], cbh)

            def _hrow(j, _):
                idr = rbh[pl.ds(j * 16, 16)]
                idc = cbh[pl.ds(j * 16, 16)]
                a = plsc.load_gather(hfl, [idr])
                b = plsc.load_gather(hfl, [idc])
                ob[pl.ds(j * 16, 16)] = a * b
                return 0
            lax.fori_loop(0, WH // 16, _hrow, 0)
            pltpu.sync_copy(ob, out.at[pl.ds(e0, WH)])


@jax.jit
def _gcn_sc(er, rowf, colf, xf, prm):
    f32 = jnp.float32
    kfn = pl.kernel(
        _body,
        out_type=(
            jax.ShapeDtypeStruct((E,), f32),
            jax.ShapeDtypeStruct((NC, NPAD), f32),
            jax.ShapeDtypeStruct((NC, NPAD, 16), f32),
            jax.ShapeDtypeStruct((NC, NPAD, 16), f32),
        ),
        mesh=plsc.VectorSubcoreMesh(core_axis_name="c", subcore_axis_name="s"),
        compiler_params=pltpu.CompilerParams(
            needs_layout_passes=False, use_tc_tiling_on_sc=False),
        scratch_types=[
            pltpu.VMEM_SHARED((NPAD,), f32),
            pltpu.VMEM_SHARED((NPAD, 16), f32),
            pltpu.VMEM_SHARED((NPAD, 16), f32),
            pltpu.VMEM_SHARED((NPAD,), f32),
            pltpu.VMEM((G, W), jnp.int32),
            pltpu.VMEM((G, W), jnp.int32),
            pltpu.VMEM((W, 16), f32),
            pltpu.VMEM((W, 16), f32),
            pltpu.VMEM((128,), f32),
            pltpu.VMEM((2 * NSL,), f32),
            pltpu.VMEM((352,), f32),
            pltpu.VMEM((NSL,), f32),
            pltpu.VMEM((NSL,), f32),
            pltpu.VMEM((NSL,), f32),
            pltpu.VMEM((NSL, 16), f32),
            pltpu.VMEM((NSL, 16), f32),
            pltpu.VMEM((NSL, 16), f32),
            pltpu.VMEM((NSL, 16), f32),
            pltpu.VMEM((NSL,), f32),
            pltpu.VMEM((NPAD,), f32),
            pltpu.VMEM((WH,), jnp.int32),
            pltpu.VMEM((WH,), jnp.int32),
            pltpu.VMEM((WH,), f32),
            pltpu.SemaphoreType.DMA,
            pltpu.SemaphoreType.DMA,
            pltpu.SemaphoreType.DMA,
            pltpu.SemaphoreType.DMA,
            pltpu.SemaphoreType.REGULAR,
        ],
    )
    return kfn(er, rowf, colf, xf, prm)


def kernel(x, edge_index, prob, W1, b1, W2, b2, Wf, bf):
    del prob
    er = edge_index.reshape(2, ROWS, W)
    rowf = edge_index[0]
    colf = edge_index[1]
    xf = x.reshape(-1)
    prm = jnp.concatenate([
        W1.reshape(-1), b1, W2.reshape(-1), b2, Wf.reshape(-1), bf,
        jnp.zeros((15,), jnp.float32),
    ])
    return _gcn_sc(er, rowf, colf, xf, prm)[0]


# speedup vs baseline: 1.0373x; 1.0373x over previous
import jax
import jax.numpy as jnp
from jax import lax
from jax.experimental import pallas as pl
from jax.experimental.pallas import tpu as pltpu
from jax.experimental.pallas import tpu_sc as plsc

N = 10000
E = 640000
NPAD = 10240
NS = 16
NC = 2
NSL = NPAD // NS
W = 125
ROWS = E // W
G = ROWS // (NS * NC)
EH = E // (NS * NC)
WH = 2000


def _body(er, ei, xf, prm, out, dex, aex1, aex2,
          deg_s, g_s, agg_s, hf_s,
          rb, cb, pay0, pay1, ones, xb, pb, degb, degc, dvb, hb, gb, ab, a2,
          hfb, hfl, rbh, cbh, ob, sem0, sem1, semc):
    sid = lax.axis_index("s")
    cid = lax.axis_index("c")
    oid = 1 - cid
    wid = sid * NC + cid
    ns0 = sid * NSL
    f32 = jnp.float32

    def bf16r(v):
        u = lax.bitcast_convert_type(v, jnp.uint32)
        r = u + jnp.uint32(0x7FFF) + ((u >> jnp.uint32(16)) & jnp.uint32(1))
        return lax.bitcast_convert_type(r & jnp.uint32(0xFFFF0000), f32)

    z16 = jnp.zeros((16,), f32)

    def _za(i, _):
        ab[i, :] = z16
        return 0
    lax.fori_loop(0, NSL, _za, 0)

    def _zb(i, _):
        hfb[pl.ds(i * 16, 16)] = z16
        return 0
    lax.fori_loop(0, NSL // 16, _zb, 0)

    one16 = jnp.ones((16,), f32)
    for j in range(128 // 16):
        ones[pl.ds(j * 16, 16)] = one16

    pltpu.sync_copy(prm, pb)

    @pl.when(sid < NS - 1)
    def _():
        pltpu.sync_copy(xf.at[pl.ds(sid * (2 * NSL), 2 * NSL)], xb)

    @pl.when(sid == NS - 1)
    def _():
        rem = 2 * N - (NS - 1) * (2 * NSL)
        pltpu.sync_copy(xf.at[pl.ds((NS - 1) * (2 * NSL), rem)],
                        xb.at[pl.ds(0, rem)])

    pltpu.sync_copy(ab, agg_s.at[pl.ds(ns0, NSL)])
    pltpu.sync_copy(hfb, deg_s.at[pl.ds(ns0, NSL)])
    plsc.subcore_barrier()

    sems = (sem0, sem1)
    with jax.named_scope("ph_hist"):
        r0 = wid * G
        pltpu.sync_copy(er.at[1, pl.ds(r0, G), :], cb)

        def _bpair(k, _):
            for b in range(2):
                i = 2 * k + b

                @pl.when(k > 0)
                def _():
                    pltpu.make_async_copy(
                        ones.at[pl.ds(0, W)], deg_s.at[cb.at[i]],
                        sems[b]).wait()
                pltpu.async_copy(ones.at[pl.ds(0, W)],
                                 deg_s.at[cb.at[i]], sems[b], add=True)
            return 0
        lax.fori_loop(0, G // 2, _bpair, 0)
        for b in range(2):
            pltpu.make_async_copy(ones.at[pl.ds(0, W)],
                                  deg_s.at[cb.at[b]], sems[b]).wait()
    plsc.subcore_barrier()
    pltpu.sync_copy(deg_s.at[pl.ds(ns0, NSL)], dex.at[cid, pl.ds(ns0, NSL)])
    pltpu.core_barrier(semc, core_axis_name="c")
    pltpu.sync_copy(dex.at[oid, pl.ds(ns0, NSL)], degc)

    pltpu.sync_copy(deg_s.at[pl.ds(ns0, NSL)], degb)
    w1r0 = bf16r(pb[pl.ds(0, 16)])
    w1r1 = bf16r(pb[pl.ds(16, 16)])

    def _rsq(i, _):
        d = degb[pl.ds(i * 16, 16)] + degc[pl.ds(i * 16, 16)] + 2.0
        ii = lax.bitcast_convert_type(d, jnp.int32)
        ii = jnp.int32(0x5F3759DF) - (ii >> 1)
        y = lax.bitcast_convert_type(ii, f32)
        y = y * (1.5 - 0.5 * d * y * y)
        y = y * (1.5 - 0.5 * d * y * y)
        y = y * (1.5 - 0.5 * d * y * y)
        y = y * (1.5 - 0.5 * d * y * y)
        dvb[pl.ds(i * 16, 16)] = y
        return 0
    lax.fori_loop(0, NSL // 16, _rsq, 0)

    def _node1(i, _):
        dv16 = dvb[pl.ds(i * 16, 16)]
        xv0 = bf16r(xb[pl.ds(i * 32, 16)])
        xv1 = bf16r(xb[pl.ds(i * 32 + 16, 16)])
        for j in range(16):
            xv = xv0 if j < 8 else xv1
            xa = xv[(2 * j) % 16]
            xc = xv[(2 * j + 1) % 16]
            h1 = xa * w1r0 + xc * w1r1
            n = i * 16 + j
            hb[n, :] = h1
            gb[n, :] = dv16[j] * h1
        return 0
    lax.fori_loop(0, NSL // 16, _node1, 0)
    pltpu.sync_copy(gb, g_s.at[pl.ds(ns0, NSL)])
    plsc.subcore_barrier()

    pays = (pay0, pay1)

    def _agg_pass():
        r0 = wid * G
        pltpu.sync_copy(er.at[0, pl.ds(r0, G), :], rb)
        pltpu.sync_copy(er.at[1, pl.ds(r0, G), :], cb)

        def _dpair(k, _):
            for b in range(2):
                i = 2 * k + b

                @pl.when(k > 0)
                def _():
                    pltpu.make_async_copy(
                        pays[b], agg_s.at[cb.at[i]], sems[b]).wait()
                pltpu.sync_copy(g_s.at[rb.at[i]], pays[b])
                pltpu.async_copy(pays[b], agg_s.at[cb.at[i]], sems[b],
                                 add=True)
            return 0
        lax.fori_loop(0, G // 2, _dpair, 0)
        for b in range(2):
            pltpu.make_async_copy(pays[b], agg_s.at[cb.at[b]],
                                  sems[b]).wait()

    with jax.named_scope("ph_agg1"):
        _agg_pass()
    plsc.subcore_barrier()
    pltpu.sync_copy(agg_s.at[pl.ds(ns0, NSL)], aex1.at[cid, pl.ds(ns0, NSL)])
    pltpu.core_barrier(semc, core_axis_name="c")
    pltpu.sync_copy(aex1.at[oid, pl.ds(ns0, NSL)], a2)

    pltpu.sync_copy(agg_s.at[pl.ds(ns0, NSL)], ab)
    b1v = pb[pl.ds(32, 16)]
    w2r = [bf16r(pb[pl.ds(48 + 16 * k, 16)]) for k in range(16)]

    def _node2(i, _):
        dv16 = dvb[pl.ds(i * 16, 16)]
        for j in range(16):
            n = i * 16 + j
            dv = dv16[j]
            z1 = jnp.maximum(
                dv * (ab[n, :] + a2[n, :]) + (2.0 * dv * dv) * hb[n, :] + b1v,
                0.0)
            ab[n, :] = z16
            z1 = bf16r(z1)
            acc = z1[0] * w2r[0]
            for k in range(1, 16):
                acc = acc + z1[k] * w2r[k]
            hb[n, :] = acc
            gb[n, :] = dv * acc
        return 0
    lax.fori_loop(0, NSL // 16, _node2, 0)
    pltpu.sync_copy(gb, g_s.at[pl.ds(ns0, NSL)])
    pltpu.sync_copy(ab, agg_s.at[pl.ds(ns0, NSL)])
    plsc.subcore_barrier()

    with jax.named_scope("ph_agg2"):
        _agg_pass()
    plsc.subcore_barrier()
    pltpu.sync_copy(agg_s.at[pl.ds(ns0, NSL)], aex2.at[cid, pl.ds(ns0, NSL)])
    pltpu.core_barrier(semc, core_axis_name="c")
    pltpu.sync_copy(aex2.at[oid, pl.ds(ns0, NSL)], a2)

    pltpu.sync_copy(agg_s.at[pl.ds(ns0, NSL)], ab)
    b2v = pb[pl.ds(304, 16)]
    wfv = bf16r(pb[pl.ds(320, 16)])
    wfs = [wfv[k] for k in range(16)]
    bf0 = pb[pl.ds(336, 16)][0]
    lane = lax.iota(jnp.int32, 16)

    def _node3(i, _):
        dv16 = dvb[pl.ds(i * 16, 16)]
        hfv = jnp.zeros((16,), f32)
        for j in range(16):
            n = i * 16 + j
            dv = dv16[j]
            z2 = jnp.maximum(
                dv * (ab[n, :] + a2[n, :]) + (2.0 * dv * dv) * hb[n, :] + b2v,
                0.0)
            z2 = bf16r(z2)
            dot = bf0
            for k in range(16):
                dot = dot + wfs[k] * z2[k]
            hfv = jnp.where(lane == j, dot, hfv)
        hfb[pl.ds(i * 16, 16)] = hfv
        return 0
    lax.fori_loop(0, NSL // 16, _node3, 0)
    pltpu.sync_copy(hfb, hf_s.at[pl.ds(ns0, NSL)])
    plsc.subcore_barrier()

    with jax.named_scope("ph_edge_out"):
        pltpu.sync_copy(hf_s, hfl)
        for gi in range(EH // WH):
            e0 = wid * EH + gi * WH
            pltpu.sync_copy(ei.at[0, pl.ds(e0, WH)], rbh)
            pltpu.sync_copy(ei.at[1, pl.ds(e0, WH)], cbh)

            def _hrow(j, _):
                idr = rbh[pl.ds(j * 16, 16)]
                idc = cbh[pl.ds(j * 16, 16)]
                a = plsc.load_gather(hfl, [idr])
                b = plsc.load_gather(hfl, [idc])
                ob[pl.ds(j * 16, 16)] = a * b
                return 0
            lax.fori_loop(0, WH // 16, _hrow, 0)
            pltpu.sync_copy(ob, out.at[pl.ds(e0, WH)])


@jax.jit
def _gcn_sc(er, ei, xf, prm):
    f32 = jnp.float32
    kfn = pl.kernel(
        _body,
        out_type=(
            jax.ShapeDtypeStruct((E,), f32),
            jax.ShapeDtypeStruct((NC, NPAD), f32),
            jax.ShapeDtypeStruct((NC, NPAD, 16), f32),
            jax.ShapeDtypeStruct((NC, NPAD, 16), f32),
        ),
        mesh=plsc.VectorSubcoreMesh(core_axis_name="c", subcore_axis_name="s"),
        compiler_params=pltpu.CompilerParams(
            needs_layout_passes=False, use_tc_tiling_on_sc=False),
        scratch_types=[
            pltpu.VMEM_SHARED((NPAD,), f32),
            pltpu.VMEM_SHARED((NPAD, 16), f32),
            pltpu.VMEM_SHARED((NPAD, 16), f32),
            pltpu.VMEM_SHARED((NPAD,), f32),
            pltpu.VMEM((G, W), jnp.int32),
            pltpu.VMEM((G, W), jnp.int32),
            pltpu.VMEM((W, 16), f32),
            pltpu.VMEM((W, 16), f32),
            pltpu.VMEM((128,), f32),
            pltpu.VMEM((2 * NSL,), f32),
            pltpu.VMEM((352,), f32),
            pltpu.VMEM((NSL,), f32),
            pltpu.VMEM((NSL,), f32),
            pltpu.VMEM((NSL,), f32),
            pltpu.VMEM((NSL, 16), f32),
            pltpu.VMEM((NSL, 16), f32),
            pltpu.VMEM((NSL, 16), f32),
            pltpu.VMEM((NSL, 16), f32),
            pltpu.VMEM((NSL,), f32),
            pltpu.VMEM((NPAD,), f32),
            pltpu.VMEM((WH,), jnp.int32),
            pltpu.VMEM((WH,), jnp.int32),
            pltpu.VMEM((WH,), f32),
            pltpu.SemaphoreType.DMA,
            pltpu.SemaphoreType.DMA,
            pltpu.SemaphoreType.REGULAR,
        ],
    )
    return kfn(er, ei, xf, prm)


def kernel(x, edge_index, prob, W1, b1, W2, b2, Wf, bf):
    del prob
    er = edge_index.reshape(2, ROWS, W)
    xf = x.reshape(-1)
    prm = jnp.concatenate([
        W1.reshape(-1), b1, W2.reshape(-1), b2, Wf.reshape(-1), bf,
        jnp.zeros((15,), jnp.float32),
    ])
    return _gcn_sc(er, edge_index, xf, prm)[0]


# speedup vs baseline: 1.1204x; 1.0801x over previous
import jax
import jax.numpy as jnp
from jax import lax
from jax.experimental import pallas as pl
from jax.experimental.pallas import tpu as pltpu
from jax.experimental.pallas import tpu_sc as plsc

N = 10000
E = 640000
NPAD = 10240
NS = 16
NC = 2
NSL = NPAD // NS
NSL2 = NPAD // (NS * NC)
W = 125
ROWS = E // W
G = ROWS // (NS * NC)
EH = E // (NS * NC)
WH = 2000


def _body(er, ei, xf, prm, out, dex, aex1, aex2, gex1, gex2, hex_,
          deg_s, g_s, agg_s,
          rb, cb, pay0, pay1, ones, xb, pb, degb, degc, dvb, hb, gb, gb2,
          ab, a2, hfb, hfl, rbh, cbh, ob, sem0, sem1, semc):
    sid = lax.axis_index("s")
    cid = lax.axis_index("c")
    oid = 1 - cid
    wid = sid * NC + cid
    ns0 = sid * NSL
    nc0 = sid * NSL + cid * NSL2
    nc0x = sid * NSL + oid * NSL2
    f32 = jnp.float32

    def bf16r(v):
        u = lax.bitcast_convert_type(v, jnp.uint32)
        r = u + jnp.uint32(0x7FFF) + ((u >> jnp.uint32(16)) & jnp.uint32(1))
        return lax.bitcast_convert_type(r & jnp.uint32(0xFFFF0000), f32)

    z16 = jnp.zeros((16,), f32)

    def _za(i, _):
        ab[i, :] = z16
        return 0
    lax.fori_loop(0, NSL2, _za, 0)

    def _zb(i, _):
        hfb[pl.ds(i * 16, 16)] = z16
        return 0
    lax.fori_loop(0, NSL // 16, _zb, 0)

    one16 = jnp.ones((16,), f32)
    for j in range(128 // 16):
        ones[pl.ds(j * 16, 16)] = one16

    pltpu.sync_copy(prm, pb)

    @pl.when(wid < NS * NC - 1)
    def _():
        pltpu.sync_copy(xf.at[pl.ds(wid * (2 * NSL2), 2 * NSL2)], xb)

    @pl.when(wid == NS * NC - 1)
    def _():
        rem = 2 * N - (NS * NC - 1) * (2 * NSL2)
        pltpu.sync_copy(xf.at[pl.ds((NS * NC - 1) * (2 * NSL2), rem)],
                        xb.at[pl.ds(0, rem)])

    pltpu.sync_copy(ab, agg_s.at[pl.ds(ns0, NSL2)])
    pltpu.sync_copy(ab, agg_s.at[pl.ds(ns0 + NSL2, NSL2)])
    pltpu.sync_copy(hfb, deg_s.at[pl.ds(ns0, NSL)])
    plsc.subcore_barrier()

    sems = (sem0, sem1)
    with jax.named_scope("ph_hist"):
        r0 = wid * G
        pltpu.sync_copy(er.at[1, pl.ds(r0, G), :], cb)

        def _bpair(k, _):
            for b in range(2):
                i = 2 * k + b

                @pl.when(k > 0)
                def _():
                    pltpu.make_async_copy(
                        ones.at[pl.ds(0, W)], deg_s.at[cb.at[i]],
                        sems[b]).wait()
                pltpu.async_copy(ones.at[pl.ds(0, W)],
                                 deg_s.at[cb.at[i]], sems[b], add=True)
            return 0
        lax.fori_loop(0, G // 2, _bpair, 0)
        for b in range(2):
            pltpu.make_async_copy(ones.at[pl.ds(0, W)],
                                  deg_s.at[cb.at[b]], sems[b]).wait()
    plsc.subcore_barrier()
    pltpu.sync_copy(deg_s.at[pl.ds(ns0, NSL)], dex.at[cid, pl.ds(ns0, NSL)])
    pltpu.core_barrier(semc, core_axis_name="c")
    pltpu.sync_copy(dex.at[oid, pl.ds(nc0, NSL2)], degc.at[pl.ds(0, NSL2)])

    pltpu.sync_copy(deg_s.at[pl.ds(nc0, NSL2)], degb.at[pl.ds(0, NSL2)])
    w1r0 = bf16r(pb[pl.ds(0, 16)])
    w1r1 = bf16r(pb[pl.ds(16, 16)])

    def _rsq(i, _):
        d = degb[pl.ds(i * 16, 16)] + degc[pl.ds(i * 16, 16)] + 2.0
        ii = lax.bitcast_convert_type(d, jnp.int32)
        ii = jnp.int32(0x5F3759DF) - (ii >> 1)
        y = lax.bitcast_convert_type(ii, f32)
        y = y * (1.5 - 0.5 * d * y * y)
        y = y * (1.5 - 0.5 * d * y * y)
        y = y * (1.5 - 0.5 * d * y * y)
        y = y * (1.5 - 0.5 * d * y * y)
        dvb[pl.ds(i * 16, 16)] = y
        return 0
    lax.fori_loop(0, NSL2 // 16, _rsq, 0)

    def _node1(i, _):
        dv16 = dvb[pl.ds(i * 16, 16)]
        xv0 = bf16r(xb[pl.ds(i * 32, 16)])
        xv1 = bf16r(xb[pl.ds(i * 32 + 16, 16)])
        for j in range(16):
            xv = xv0 if j < 8 else xv1
            xa = xv[(2 * j) % 16]
            xc = xv[(2 * j + 1) % 16]
            h1 = xa * w1r0 + xc * w1r1
            n = i * 16 + j
            hb[n, :] = h1
            gb[n, :] = dv16[j] * h1
        return 0
    lax.fori_loop(0, NSL2 // 16, _node1, 0)
    pltpu.sync_copy(gb, g_s.at[pl.ds(nc0, NSL2)])
    pltpu.sync_copy(gb, gex1.at[pl.ds(nc0, NSL2)])
    pltpu.core_barrier(semc, core_axis_name="c")
    pltpu.sync_copy(gex1.at[pl.ds(nc0x, NSL2)], gb2)
    pltpu.sync_copy(gb2, g_s.at[pl.ds(nc0x, NSL2)])
    plsc.subcore_barrier()

    pays = (pay0, pay1)

    def _agg_pass():
        r0 = wid * G
        pltpu.sync_copy(er.at[0, pl.ds(r0, G), :], rb)
        pltpu.sync_copy(er.at[1, pl.ds(r0, G), :], cb)

        def _dpair(k, _):
            for b in range(2):
                i = 2 * k + b

                @pl.when(k > 0)
                def _():
                    pltpu.make_async_copy(
                        pays[b], agg_s.at[cb.at[i]], sems[b]).wait()
                pltpu.sync_copy(g_s.at[rb.at[i]], pays[b])
                pltpu.async_copy(pays[b], agg_s.at[cb.at[i]], sems[b],
                                 add=True)
            return 0
        lax.fori_loop(0, G // 2, _dpair, 0)
        for b in range(2):
            pltpu.make_async_copy(pays[b], agg_s.at[cb.at[b]],
                                  sems[b]).wait()

    with jax.named_scope("ph_agg1"):
        _agg_pass()
    plsc.subcore_barrier()
    pltpu.sync_copy(agg_s.at[pl.ds(ns0, NSL)], aex1.at[cid, pl.ds(ns0, NSL)])
    pltpu.core_barrier(semc, core_axis_name="c")
    pltpu.sync_copy(aex1.at[oid, pl.ds(nc0, NSL2)], a2)

    pltpu.sync_copy(agg_s.at[pl.ds(nc0, NSL2)], ab)
    b1v = pb[pl.ds(32, 16)]
    w2r = [bf16r(pb[pl.ds(48 + 16 * k, 16)]) for k in range(16)]

    def _node2(i, _):
        dv16 = dvb[pl.ds(i * 16, 16)]
        for j in range(16):
            n = i * 16 + j
            dv = dv16[j]
            z1 = jnp.maximum(
                dv * (ab[n, :] + a2[n, :]) + (2.0 * dv * dv) * hb[n, :] + b1v,
                0.0)
            ab[n, :] = z16
            z1 = bf16r(z1)
            acc = z1[0] * w2r[0]
            for k in range(1, 16):
                acc = acc + z1[k] * w2r[k]
            hb[n, :] = acc
            gb[n, :] = dv * acc
        return 0
    lax.fori_loop(0, NSL2 // 16, _node2, 0)
    pltpu.sync_copy(gb, g_s.at[pl.ds(nc0, NSL2)])
    pltpu.sync_copy(gb, gex2.at[pl.ds(nc0, NSL2)])
    pltpu.sync_copy(ab, agg_s.at[pl.ds(ns0, NSL2)])
    pltpu.sync_copy(ab, agg_s.at[pl.ds(ns0 + NSL2, NSL2)])
    pltpu.core_barrier(semc, core_axis_name="c")
    pltpu.sync_copy(gex2.at[pl.ds(nc0x, NSL2)], gb2)
    pltpu.sync_copy(gb2, g_s.at[pl.ds(nc0x, NSL2)])
    plsc.subcore_barrier()

    with jax.named_scope("ph_agg2"):
        _agg_pass()
    plsc.subcore_barrier()
    pltpu.sync_copy(agg_s.at[pl.ds(ns0, NSL)], aex2.at[cid, pl.ds(ns0, NSL)])
    pltpu.core_barrier(semc, core_axis_name="c")
    pltpu.sync_copy(aex2.at[oid, pl.ds(nc0, NSL2)], a2)

    pltpu.sync_copy(agg_s.at[pl.ds(nc0, NSL2)], ab)
    b2v = pb[pl.ds(304, 16)]
    wfv = bf16r(pb[pl.ds(320, 16)])
    wfs = [wfv[k] for k in range(16)]
    bf0 = pb[pl.ds(336, 16)][0]
    lane = lax.iota(jnp.int32, 16)

    def _node3(i, _):
        dv16 = dvb[pl.ds(i * 16, 16)]
        hfv = jnp.zeros((16,), f32)
        for j in range(16):
            n = i * 16 + j
            dv = dv16[j]
            z2 = jnp.maximum(
                dv * (ab[n, :] + a2[n, :]) + (2.0 * dv * dv) * hb[n, :] + b2v,
                0.0)
            z2 = bf16r(z2)
            dot = bf0
            for k in range(16):
                dot = dot + wfs[k] * z2[k]
            hfv = jnp.where(lane == j, dot, hfv)
        hfb[pl.ds(i * 16, 16)] = hfv
        return 0
    lax.fori_loop(0, NSL2 // 16, _node3, 0)
    pltpu.sync_copy(hfb.at[pl.ds(0, NSL2)], hex_.at[pl.ds(nc0, NSL2)])
    plsc.subcore_barrier()
    pltpu.core_barrier(semc, core_axis_name="c")

    with jax.named_scope("ph_edge_out"):
        pltpu.sync_copy(hex_, hfl)
        for gi in range(EH // WH):
            e0 = wid * EH + gi * WH
            pltpu.sync_copy(ei.at[0, pl.ds(e0, WH)], rbh)
            pltpu.sync_copy(ei.at[1, pl.ds(e0, WH)], cbh)

            def _hrow(j, _):
                idr = rbh[pl.ds(j * 16, 16)]
                idc = cbh[pl.ds(j * 16, 16)]
                a = plsc.load_gather(hfl, [idr])
                b = plsc.load_gather(hfl, [idc])
                ob[pl.ds(j * 16, 16)] = a * b
                return 0
            lax.fori_loop(0, WH // 16, _hrow, 0)
            pltpu.sync_copy(ob, out.at[pl.ds(e0, WH)])


@jax.jit
def _gcn_sc(er, ei, xf, prm):
    f32 = jnp.float32
    kfn = pl.kernel(
        _body,
        out_type=(
            jax.ShapeDtypeStruct((E,), f32),
            jax.ShapeDtypeStruct((NC, NPAD), f32),
            jax.ShapeDtypeStruct((NC, NPAD, 16), f32),
            jax.ShapeDtypeStruct((NC, NPAD, 16), f32),
            jax.ShapeDtypeStruct((NPAD, 16), f32),
            jax.ShapeDtypeStruct((NPAD, 16), f32),
            jax.ShapeDtypeStruct((NPAD,), f32),
        ),
        mesh=plsc.VectorSubcoreMesh(core_axis_name="c", subcore_axis_name="s"),
        compiler_params=pltpu.CompilerParams(
            needs_layout_passes=False, use_tc_tiling_on_sc=False),
        scratch_types=[
            pltpu.VMEM_SHARED((NPAD,), f32),
            pltpu.VMEM_SHARED((NPAD, 16), f32),
            pltpu.VMEM_SHARED((NPAD, 16), f32),
            pltpu.VMEM((G, W), jnp.int32),
            pltpu.VMEM((G, W), jnp.int32),
            pltpu.VMEM((W, 16), f32),
            pltpu.VMEM((W, 16), f32),
            pltpu.VMEM((128,), f32),
            pltpu.VMEM((2 * NSL2,), f32),
            pltpu.VMEM((352,), f32),
            pltpu.VMEM((NSL2,), f32),
            pltpu.VMEM((NSL2,), f32),
            pltpu.VMEM((NSL2,), f32),
            pltpu.VMEM((NSL2, 16), f32),
            pltpu.VMEM((NSL2, 16), f32),
            pltpu.VMEM((NSL2, 16), f32),
            pltpu.VMEM((NSL2, 16), f32),
            pltpu.VMEM((NSL2, 16), f32),
            pltpu.VMEM((NSL,), f32),
            pltpu.VMEM((NPAD,), f32),
            pltpu.VMEM((WH,), jnp.int32),
            pltpu.VMEM((WH,), jnp.int32),
            pltpu.VMEM((WH,), f32),
            pltpu.SemaphoreType.DMA,
            pltpu.SemaphoreType.DMA,
            pltpu.SemaphoreType.REGULAR,
        ],
    )
    return kfn(er, ei, xf, prm)


def kernel(x, edge_index, prob, W1, b1, W2, b2, Wf, bf):
    del prob
    er = edge_index.reshape(2, ROWS, W)
    xf = x.reshape(-1)
    prm = jnp.concatenate([
        W1.reshape(-1), b1, W2.reshape(-1), b2, Wf.reshape(-1), bf,
        jnp.zeros((15,), jnp.float32),
    ])
    return _gcn_sc(er, edge_index, xf, prm)[0]


# speedup vs baseline: 1.1418x; 1.0191x over previous
import jax
import jax.numpy as jnp
from jax import lax
from jax.experimental import pallas as pl
from jax.experimental.pallas import tpu as pltpu
from jax.experimental.pallas import tpu_sc as plsc

N = 10000
E = 640000
NPAD = 10240
NS = 16
NC = 2
NSL = NPAD // NS
NSL2 = NPAD // (NS * NC)
W = 125
ROWS = E // W
G = ROWS // (NS * NC)
EH = E // (NS * NC)
WH = 2000


def _body(er, ei, xf, prm, out, dex, aex1, aex2, gex1, gex2, hex_,
          deg_s, g_s, agg_s,
          rb, cb, pay0, pay1, ones, xb, pb, degb, degc, dvb, hb, gb, gb2,
          ab, a2, hfb, hfl, rbh, cbh, ob, sem0, sem1, semc):
    sid = lax.axis_index("s")
    cid = lax.axis_index("c")
    oid = 1 - cid
    wid = sid * NC + cid
    ns0 = sid * NSL
    nc0 = sid * NSL + cid * NSL2
    nc0x = sid * NSL + oid * NSL2
    f32 = jnp.float32

    def bf16r(v):
        u = lax.bitcast_convert_type(v, jnp.uint32)
        r = u + jnp.uint32(0x7FFF) + ((u >> jnp.uint32(16)) & jnp.uint32(1))
        return lax.bitcast_convert_type(r & jnp.uint32(0xFFFF0000), f32)

    z16 = jnp.zeros((16,), f32)

    def _za(i, _):
        ab[i, :] = z16
        return 0
    lax.fori_loop(0, NSL2, _za, 0)

    def _zb(i, _):
        hfb[pl.ds(i * 16, 16)] = z16
        return 0
    lax.fori_loop(0, NSL // 16, _zb, 0)

    one16 = jnp.ones((16,), f32)
    for j in range(128 // 16):
        ones[pl.ds(j * 16, 16)] = one16

    pltpu.sync_copy(prm, pb)

    @pl.when(wid < NS * NC - 1)
    def _():
        pltpu.sync_copy(xf.at[pl.ds(wid * (2 * NSL2), 2 * NSL2)], xb)

    @pl.when(wid == NS * NC - 1)
    def _():
        rem = 2 * N - (NS * NC - 1) * (2 * NSL2)
        pltpu.sync_copy(xf.at[pl.ds((NS * NC - 1) * (2 * NSL2), rem)],
                        xb.at[pl.ds(0, rem)])

    pltpu.sync_copy(ab, agg_s.at[pl.ds(ns0, NSL2)])
    pltpu.sync_copy(ab, agg_s.at[pl.ds(ns0 + NSL2, NSL2)])
    pltpu.sync_copy(hfb, deg_s.at[pl.ds(ns0, NSL)])
    pltpu.sync_copy(er.at[0, pl.ds(wid * G, G), :], rb)
    pltpu.sync_copy(er.at[1, pl.ds(wid * G, G), :], cb)
    plsc.subcore_barrier()

    sems = (sem0, sem1)
    with jax.named_scope("ph_hist"):
        def _bpair(k, _):
            for b in range(2):
                i = 2 * k + b

                @pl.when(k > 0)
                def _():
                    pltpu.make_async_copy(
                        ones.at[pl.ds(0, W)], deg_s.at[cb.at[i]],
                        sems[b]).wait()
                pltpu.async_copy(ones.at[pl.ds(0, W)],
                                 deg_s.at[cb.at[i]], sems[b], add=True)
            return 0
        lax.fori_loop(0, G // 2, _bpair, 0)
        for b in range(2):
            pltpu.make_async_copy(ones.at[pl.ds(0, W)],
                                  deg_s.at[cb.at[b]], sems[b]).wait()
    plsc.subcore_barrier()
    pltpu.sync_copy(deg_s.at[pl.ds(ns0, NSL)], dex.at[cid, pl.ds(ns0, NSL)])
    pltpu.core_barrier(semc, core_axis_name="c")
    pltpu.sync_copy(dex.at[oid, pl.ds(nc0, NSL2)], degc.at[pl.ds(0, NSL2)])

    pltpu.sync_copy(deg_s.at[pl.ds(nc0, NSL2)], degb.at[pl.ds(0, NSL2)])
    w1r0 = bf16r(pb[pl.ds(0, 16)])
    w1r1 = bf16r(pb[pl.ds(16, 16)])

    def _rsq(i, _):
        d = degb[pl.ds(i * 16, 16)] + degc[pl.ds(i * 16, 16)] + 2.0
        ii = lax.bitcast_convert_type(d, jnp.int32)
        ii = jnp.int32(0x5F3759DF) - (ii >> 1)
        y = lax.bitcast_convert_type(ii, f32)
        y = y * (1.5 - 0.5 * d * y * y)
        y = y * (1.5 - 0.5 * d * y * y)
        y = y * (1.5 - 0.5 * d * y * y)
        y = y * (1.5 - 0.5 * d * y * y)
        dvb[pl.ds(i * 16, 16)] = y
        return 0
    lax.fori_loop(0, NSL2 // 16, _rsq, 0)

    def _node1(i, _):
        dv16 = dvb[pl.ds(i * 16, 16)]
        xv0 = bf16r(xb[pl.ds(i * 32, 16)])
        xv1 = bf16r(xb[pl.ds(i * 32 + 16, 16)])
        for j in range(16):
            xv = xv0 if j < 8 else xv1
            xa = xv[(2 * j) % 16]
            xc = xv[(2 * j + 1) % 16]
            h1 = xa * w1r0 + xc * w1r1
            n = i * 16 + j
            hb[n, :] = h1
            gb[n, :] = dv16[j] * h1
        return 0
    lax.fori_loop(0, NSL2 // 16, _node1, 0)
    pltpu.sync_copy(gb, g_s.at[pl.ds(nc0, NSL2)])
    pltpu.sync_copy(gb, gex1.at[pl.ds(nc0, NSL2)])
    pltpu.core_barrier(semc, core_axis_name="c")
    pltpu.sync_copy(gex1.at[pl.ds(nc0x, NSL2)], gb2)
    pltpu.sync_copy(gb2, g_s.at[pl.ds(nc0x, NSL2)])
    plsc.subcore_barrier()

    pays = (pay0, pay1)

    def _agg_pass():
        def _dpair(k, _):
            for b in range(2):
                i = 2 * k + b

                @pl.when(k > 0)
                def _():
                    pltpu.make_async_copy(
                        pays[b], agg_s.at[cb.at[i]], sems[b]).wait()
                pltpu.sync_copy(g_s.at[rb.at[i]], pays[b])
                pltpu.async_copy(pays[b], agg_s.at[cb.at[i]], sems[b],
                                 add=True)
            return 0
        lax.fori_loop(0, G // 2, _dpair, 0)
        for b in range(2):
            pltpu.make_async_copy(pays[b], agg_s.at[cb.at[b]],
                                  sems[b]).wait()

    with jax.named_scope("ph_agg1"):
        _agg_pass()
    plsc.subcore_barrier()
    pltpu.sync_copy(agg_s.at[pl.ds(ns0, NSL)], aex1.at[cid, pl.ds(ns0, NSL)])
    pltpu.core_barrier(semc, core_axis_name="c")
    pltpu.sync_copy(aex1.at[oid, pl.ds(nc0, NSL2)], a2)

    pltpu.sync_copy(agg_s.at[pl.ds(nc0, NSL2)], ab)
    b1v = pb[pl.ds(32, 16)]
    w2r = [bf16r(pb[pl.ds(48 + 16 * k, 16)]) for k in range(16)]

    def _node2(i, _):
        dv16 = dvb[pl.ds(i * 16, 16)]
        for j in range(16):
            n = i * 16 + j
            dv = dv16[j]
            z1 = jnp.maximum(
                dv * (ab[n, :] + a2[n, :]) + (2.0 * dv * dv) * hb[n, :] + b1v,
                0.0)
            ab[n, :] = z16
            z1 = bf16r(z1)
            acc = z1[0] * w2r[0]
            for k in range(1, 16):
                acc = acc + z1[k] * w2r[k]
            hb[n, :] = acc
            gb[n, :] = dv * acc
        return 0
    lax.fori_loop(0, NSL2 // 16, _node2, 0)
    pltpu.sync_copy(gb, g_s.at[pl.ds(nc0, NSL2)])
    pltpu.sync_copy(gb, gex2.at[pl.ds(nc0, NSL2)])
    pltpu.sync_copy(ab, agg_s.at[pl.ds(ns0, NSL2)])
    pltpu.sync_copy(ab, agg_s.at[pl.ds(ns0 + NSL2, NSL2)])
    pltpu.core_barrier(semc, core_axis_name="c")
    pltpu.sync_copy(gex2.at[pl.ds(nc0x, NSL2)], gb2)
    pltpu.sync_copy(gb2, g_s.at[pl.ds(nc0x, NSL2)])
    plsc.subcore_barrier()

    with jax.named_scope("ph_agg2"):
        _agg_pass()
    plsc.subcore_barrier()
    pltpu.sync_copy(agg_s.at[pl.ds(ns0, NSL)], aex2.at[cid, pl.ds(ns0, NSL)])
    pltpu.core_barrier(semc, core_axis_name="c")
    pltpu.sync_copy(aex2.at[oid, pl.ds(nc0, NSL2)], a2)

    pltpu.sync_copy(agg_s.at[pl.ds(nc0, NSL2)], ab)
    b2v = pb[pl.ds(304, 16)]
    wfv = bf16r(pb[pl.ds(320, 16)])
    wfs = [wfv[k] for k in range(16)]
    bf0 = pb[pl.ds(336, 16)][0]
    lane = lax.iota(jnp.int32, 16)

    def _node3(i, _):
        dv16 = dvb[pl.ds(i * 16, 16)]
        hfv = jnp.zeros((16,), f32)
        for j in range(16):
            n = i * 16 + j
            dv = dv16[j]
            z2 = jnp.maximum(
                dv * (ab[n, :] + a2[n, :]) + (2.0 * dv * dv) * hb[n, :] + b2v,
                0.0)
            z2 = bf16r(z2)
            dot = bf0
            for k in range(16):
                dot = dot + wfs[k] * z2[k]
            hfv = jnp.where(lane == j, dot, hfv)
        hfb[pl.ds(i * 16, 16)] = hfv
        return 0
    lax.fori_loop(0, NSL2 // 16, _node3, 0)
    pltpu.sync_copy(hfb.at[pl.ds(0, NSL2)], hex_.at[pl.ds(nc0, NSL2)])
    plsc.subcore_barrier()
    pltpu.core_barrier(semc, core_axis_name="c")

    with jax.named_scope("ph_edge_out"):
        pltpu.sync_copy(hex_, hfl)
        for gi in range(EH // WH):
            e0 = wid * EH + gi * WH
            pltpu.sync_copy(ei.at[0, pl.ds(e0, WH)], rbh)
            pltpu.sync_copy(ei.at[1, pl.ds(e0, WH)], cbh)

            def _hrow(j, _):
                idr = rbh[pl.ds(j * 16, 16)]
                idc = cbh[pl.ds(j * 16, 16)]
                a = plsc.load_gather(hfl, [idr])
                b = plsc.load_gather(hfl, [idc])
                ob[pl.ds(j * 16, 16)] = a * b
                return 0
            lax.fori_loop(0, WH // 16, _hrow, 0)
            pltpu.sync_copy(ob, out.at[pl.ds(e0, WH)])


@jax.jit
def _gcn_sc(er, ei, xf, prm):
    f32 = jnp.float32
    kfn = pl.kernel(
        _body,
        out_type=(
            jax.ShapeDtypeStruct((E,), f32),
            jax.ShapeDtypeStruct((NC, NPAD), f32),
            jax.ShapeDtypeStruct((NC, NPAD, 16), f32),
            jax.ShapeDtypeStruct((NC, NPAD, 16), f32),
            jax.ShapeDtypeStruct((NPAD, 16), f32),
            jax.ShapeDtypeStruct((NPAD, 16), f32),
            jax.ShapeDtypeStruct((NPAD,), f32),
        ),
        mesh=plsc.VectorSubcoreMesh(core_axis_name="c", subcore_axis_name="s"),
        compiler_params=pltpu.CompilerParams(
            needs_layout_passes=False, use_tc_tiling_on_sc=False),
        scratch_types=[
            pltpu.VMEM_SHARED((NPAD,), f32),
            pltpu.VMEM_SHARED((NPAD, 16), f32),
            pltpu.VMEM_SHARED((NPAD, 16), f32),
            pltpu.VMEM((G, W), jnp.int32),
            pltpu.VMEM((G, W), jnp.int32),
            pltpu.VMEM((W, 16), f32),
            pltpu.VMEM((W, 16), f32),
            pltpu.VMEM((128,), f32),
            pltpu.VMEM((2 * NSL2,), f32),
            pltpu.VMEM((352,), f32),
            pltpu.VMEM((NSL2,), f32),
            pltpu.VMEM((NSL2,), f32),
            pltpu.VMEM((NSL2,), f32),
            pltpu.VMEM((NSL2, 16), f32),
            pltpu.VMEM((NSL2, 16), f32),
            pltpu.VMEM((NSL2, 16), f32),
            pltpu.VMEM((NSL2, 16), f32),
            pltpu.VMEM((NSL2, 16), f32),
            pltpu.VMEM((NSL,), f32),
            pltpu.VMEM((NPAD,), f32),
            pltpu.VMEM((WH,), jnp.int32),
            pltpu.VMEM((WH,), jnp.int32),
            pltpu.VMEM((WH,), f32),
            pltpu.SemaphoreType.DMA,
            pltpu.SemaphoreType.DMA,
            pltpu.SemaphoreType.REGULAR,
        ],
    )
    return kfn(er, ei, xf, prm)


def kernel(x, edge_index, prob, W1, b1, W2, b2, Wf, bf):
    del prob
    er = edge_index.reshape(2, ROWS, W)
    xf = x.reshape(-1)
    prm = jnp.concatenate([
        W1.reshape(-1), b1, W2.reshape(-1), b2, Wf.reshape(-1), bf,
        jnp.zeros((15,), jnp.float32),
    ])
    return _gcn_sc(er, edge_index, xf, prm)[0]
